# Initial kernel scaffold; baseline (speedup 1.0000x reference)
import jax
import jax.numpy as jnp
from jax.experimental import pallas as pl


def kernel(x, emb_weight):
    raise NotImplementedError("write your pallas kernel here")


# speedup vs baseline: 4.8092x; 4.8092x over previous
import functools

import jax
import jax.numpy as jnp
from jax import lax
from jax.experimental import pallas as pl
from jax.experimental.pallas import tpu as pltpu
from jax.experimental.pallas import tpu_sc as plsc

NTOKEN = 1000000
EMB_DIM = 32
BATCH = 16384
HIST = 200

B_TOTAL = BATCH * HIST
NC = 2
NS = 16
NW = NC * NS
B_PER_W = B_TOTAL // NW
CHUNK = 1024
N_CHUNKS = B_PER_W // CHUNK


def _make_kernel():
    mesh = plsc.VectorSubcoreMesh(core_axis_name="c", subcore_axis_name="s")

    @functools.partial(
        pl.kernel,
        mesh=mesh,
        out_type=jax.ShapeDtypeStruct((B_TOTAL, EMB_DIM), jnp.float32),
        scratch_types=[
            pltpu.VMEM((CHUNK,), jnp.int32),
            pltpu.VMEM((CHUNK, EMB_DIM), jnp.float32),
            pltpu.SemaphoreType.DMA,
        ],
        compiler_params=pltpu.CompilerParams(use_tc_tiling_on_sc=False),
    )
    def emb_kernel(idx_hbm, table_hbm, out_hbm, idx_v, rows_v, sem):
        wid = lax.axis_index("s") * NC + lax.axis_index("c")
        wbase = wid * B_PER_W

        def body(ci, carry):
            base = wbase + ci * CHUNK
            pltpu.sync_copy(idx_hbm.at[pl.ds(base, CHUNK)], idx_v)
            pltpu.async_copy(table_hbm.at[idx_v], rows_v, sem).wait()
            pltpu.sync_copy(rows_v, out_hbm.at[pl.ds(base, CHUNK)])
            return carry

        lax.fori_loop(0, N_CHUNKS, body, 0)

    return emb_kernel


_emb_kernel = _make_kernel()


@jax.jit
def kernel(x, emb_weight):
    idx = x.reshape(-1).astype(jnp.int32)
    out = _emb_kernel(idx, emb_weight)
    return out.reshape(BATCH, HIST, EMB_DIM)


# speedup vs baseline: 5.0096x; 1.0417x over previous
import functools

import jax
import jax.numpy as jnp
from jax import lax
from jax.experimental import pallas as pl
from jax.experimental.pallas import tpu as pltpu
from jax.experimental.pallas import tpu_sc as plsc

NTOKEN = 1000000
EMB_DIM = 32
BATCH = 16384
HIST = 200

B_TOTAL = BATCH * HIST
NC = 2
NS = 16
NW = NC * NS
B_PER_W = B_TOTAL // NW
CHUNK = 1024
N_CHUNKS = B_PER_W // CHUNK
S = 10
NSUPER = N_CHUNKS // S


def _make_kernel():
    mesh = plsc.VectorSubcoreMesh(core_axis_name="c", subcore_axis_name="s")

    @functools.partial(
        pl.kernel,
        mesh=mesh,
        out_type=jax.ShapeDtypeStruct((B_TOTAL, EMB_DIM), jnp.float32),
        scratch_types=[
            pltpu.VMEM((S * CHUNK,), jnp.int32),
            pltpu.VMEM((CHUNK, EMB_DIM), jnp.float32),
            pltpu.VMEM((CHUNK, EMB_DIM), jnp.float32),
            pltpu.SemaphoreType.DMA,
            pltpu.SemaphoreType.DMA,
            pltpu.SemaphoreType.DMA,
            pltpu.SemaphoreType.DMA,
        ],
        compiler_params=pltpu.CompilerParams(use_tc_tiling_on_sc=False),
    )
    def emb_kernel(idx_hbm, table_hbm, out_hbm, idx_v, rows0, rows1,
                   sem_g0, sem_g1, sem_s0, sem_s1):
        wid = lax.axis_index("s") * NC + lax.axis_index("c")
        wbase = wid * B_PER_W
        rows = (rows0, rows1)
        sg = (sem_g0, sem_g1)
        ss = (sem_s0, sem_s1)

        def sb_body(si, carry):
            sbase = wbase + si * (S * CHUNK)
            pltpu.sync_copy(idx_hbm.at[pl.ds(sbase, S * CHUNK)], idx_v)
            gh = [None, None]
            sh = [None, None]
            gh[0] = pltpu.async_copy(
                table_hbm.at[idx_v.at[pl.ds(0, CHUNK)]], rows[0], sg[0])
            for j in range(S):
                b = j & 1
                gh[b].wait()
                if j + 1 < S:
                    if j >= 1:
                        sh[1 - b].wait()
                    gh[1 - b] = pltpu.async_copy(
                        table_hbm.at[idx_v.at[pl.ds((j + 1) * CHUNK, CHUNK)]],
                        rows[1 - b], sg[1 - b])
                sh[b] = pltpu.async_copy(
                    rows[b], out_hbm.at[pl.ds(sbase + j * CHUNK, CHUNK)], ss[b])
            sh[0].wait()
            sh[1].wait()
            return carry

        lax.fori_loop(0, NSUPER, sb_body, 0)

    return emb_kernel


_emb_kernel = _make_kernel()


@jax.jit
def kernel(x, emb_weight):
    idx = x.reshape(-1).astype(jnp.int32)
    out = _emb_kernel(idx, emb_weight)
    return out.reshape(BATCH, HIST, EMB_DIM)
